# baseline (speedup 1.0000x reference)
import jax
import jax.numpy as jnp
from jax import lax
from jax.experimental import pallas as pl
from jax.experimental.pallas import tpu as pltpu
from jax.experimental.pallas import tpu_sc as plsc

DIM = 64
EPS = 1e-5
NC, NS = 2, 16
NW = NC * NS
CHUNK = 128


def _sc_gather_add(i0, i1, t0, t1, n):
    per_w = n // NW
    chunks = per_w // CHUNK
    mesh = plsc.VectorSubcoreMesh(core_axis_name="c", subcore_axis_name="s")

    def body(i0_hbm, i1_hbm, t0_hbm, t1_hbm, out_hbm,
             i0_v, i1_v, r0_v, r1_v, sem0, sem1):
        wid = lax.axis_index("s") * NC + lax.axis_index("c")
        base = wid * per_w

        def chunk_body(j, carry):
            off = base + j * CHUNK
            pltpu.sync_copy(i0_hbm.at[pl.ds(off, CHUNK)], i0_v)
            pltpu.sync_copy(i1_hbm.at[pl.ds(off, CHUNK)], i1_v)
            cp0 = pltpu.async_copy(t0_hbm.at[i0_v], r0_v, sem0)
            cp1 = pltpu.async_copy(t1_hbm.at[i1_v], r1_v, sem1)
            cp0.wait()
            cp1.wait()

            def add_row(k, carry2):
                for q in range(DIM // 16):
                    sl = pl.ds(q * 16, 16)
                    r0_v[k, sl] = r0_v[k, sl] + r1_v[k, sl]
                return carry2

            lax.fori_loop(0, CHUNK, add_row, 0, unroll=4)
            pltpu.sync_copy(r0_v, out_hbm.at[pl.ds(off, CHUNK)])
            return carry

        lax.fori_loop(0, chunks, chunk_body, 0)

    f = pl.kernel(
        body,
        out_type=jax.ShapeDtypeStruct((n, DIM), jnp.float32),
        mesh=mesh,
        scratch_types=[
            pltpu.VMEM((CHUNK,), jnp.int32),
            pltpu.VMEM((CHUNK,), jnp.int32),
            pltpu.VMEM((CHUNK, DIM), jnp.float32),
            pltpu.VMEM((CHUNK, DIM), jnp.float32),
            pltpu.SemaphoreType.DMA,
            pltpu.SemaphoreType.DMA,
        ],
        compiler_params=pltpu.CompilerParams(use_tc_tiling_on_sc=False),
    )
    return f(i0, i1, t0, t1)


def _tc_layernorm(emb, gamma, beta):
    n = emb.shape[0]
    blk = 512

    def body(e_ref, g_ref, b_ref, o_ref):
        x = e_ref[...]
        mu = jnp.mean(x, axis=-1, keepdims=True)
        xc = x - mu
        var = jnp.mean(xc * xc, axis=-1, keepdims=True)
        o_ref[...] = xc * lax.rsqrt(var + EPS) * g_ref[...] + b_ref[...]

    return pl.pallas_call(
        body,
        grid=(n // blk,),
        in_specs=[
            pl.BlockSpec((blk, DIM), lambda i: (i, 0)),
            pl.BlockSpec((1, DIM), lambda i: (0, 0)),
            pl.BlockSpec((1, DIM), lambda i: (0, 0)),
        ],
        out_specs=pl.BlockSpec((blk, DIM), lambda i: (i, 0)),
        out_shape=jax.ShapeDtypeStruct((n, DIM), jnp.float32),
    )(emb, gamma, beta)


def kernel(idx0, idx1, table0, table1, gamma, beta):
    b, f = idx0.shape
    n = b * f
    i0 = idx0.reshape(n).astype(jnp.int32)
    i1 = idx1.reshape(n).astype(jnp.int32)
    emb = _sc_gather_add(i0, i1, table0, table1, n)
    out = _tc_layernorm(emb, gamma.reshape(1, DIM), beta.reshape(1, DIM))
    return out.reshape(b, f, DIM)


# speedup vs baseline: 1.3801x; 1.3801x over previous
import jax
import jax.numpy as jnp
from jax import lax
from jax.experimental import pallas as pl
from jax.experimental.pallas import tpu as pltpu
from jax.experimental.pallas import tpu_sc as plsc

DIM = 64
EPS = 1e-5
NC, NS = 2, 16
NW = NC * NS
CHUNK = 128
NQ = DIM // 16


def _rsqrt_newton(x):
    i = lax.bitcast_convert_type(x, jnp.int32)
    i = jnp.int32(0x5F3759DF) - (i >> 1)
    y = lax.bitcast_convert_type(i, jnp.float32)
    for _ in range(3):
        y = y * (1.5 - 0.5 * x * y * y)
    return y


def _fused_sc(i0, i1, t0, t1, gamma, beta, n):
    per_w = n // NW
    chunks = per_w // CHUNK
    assert chunks % 2 == 0
    mesh = plsc.VectorSubcoreMesh(core_axis_name="c", subcore_axis_name="s")

    def body(i0_hbm, i1_hbm, t0_hbm, t1_hbm, g_hbm, b_hbm, out_hbm,
             i0_v, i1_v, gb_v,
             r0a, r1a, r0b, r1b, oa, ob,
             sga, sgb, soa, sob):
        wid = lax.axis_index("s") * NC + lax.axis_index("c")
        base = wid * per_w
        irow = wid * chunks

        pltpu.sync_copy(i0_hbm.at[pl.ds(irow, chunks)], i0_v)
        pltpu.sync_copy(i1_hbm.at[pl.ds(irow, chunks)], i1_v)
        pltpu.sync_copy(g_hbm, gb_v.at[0])
        pltpu.sync_copy(b_hbm, gb_v.at[1])
        gv = [gb_v[0, pl.ds(16 * q, 16)] for q in range(NQ)]
        bv = [gb_v[1, pl.ds(16 * q, 16)] for q in range(NQ)]

        def issue_gathers(j, r0x, r1x, sgx):
            pltpu.async_copy(t0_hbm.at[i0_v.at[j]], r0x, sgx)
            pltpu.async_copy(t1_hbm.at[i1_v.at[j]], r1x, sgx)

        def wait_gathers(r0x, r1x, sgx):
            pltpu.make_async_copy(t0_hbm.at[i0_v.at[0]], r0x, sgx).wait()
            pltpu.make_async_copy(t1_hbm.at[i1_v.at[0]], r1x, sgx).wait()

        def out_dst(j):
            return out_hbm.at[pl.ds(base + j * CHUNK, CHUNK)]

        def compute(r0x, r1x, ox):
            def row(k, carry):
                a = [r0x[k, pl.ds(16 * q, 16)] + r1x[k, pl.ds(16 * q, 16)]
                     for q in range(NQ)]
                tot = jnp.sum((a[0] + a[1]) + (a[2] + a[3]))
                tot2 = jnp.sum((a[0] * a[0] + a[1] * a[1])
                               + (a[2] * a[2] + a[3] * a[3]))
                mu = tot * (1.0 / DIM)
                var = tot2 * (1.0 / DIM) - mu * mu
                rstd = _rsqrt_newton(var + EPS)
                for q in range(NQ):
                    ox[k, pl.ds(16 * q, 16)] = (a[q] - mu) * (rstd * gv[q]) + bv[q]
                return carry
            lax.fori_loop(0, CHUNK, row, 0, unroll=2)

        issue_gathers(0, r0a, r1a, sga)
        pltpu.async_copy(oa, out_dst(0), soa)
        pltpu.async_copy(ob, out_dst(1), sob)

        def pair(p, carry):
            ja = 2 * p
            wait_gathers(r0a, r1a, sga)
            issue_gathers(ja + 1, r0b, r1b, sgb)
            pltpu.make_async_copy(oa, out_dst(0), soa).wait()
            compute(r0a, r1a, oa)
            pltpu.async_copy(oa, out_dst(ja), soa)
            wait_gathers(r0b, r1b, sgb)

            @pl.when(p < chunks // 2 - 1)
            def _():
                issue_gathers(ja + 2, r0a, r1a, sga)

            pltpu.make_async_copy(ob, out_dst(0), sob).wait()
            compute(r0b, r1b, ob)
            pltpu.async_copy(ob, out_dst(ja + 1), sob)
            return carry

        lax.fori_loop(0, chunks // 2, pair, 0)
        pltpu.make_async_copy(oa, out_dst(0), soa).wait()
        pltpu.make_async_copy(ob, out_dst(0), sob).wait()

    f = pl.kernel(
        body,
        out_type=jax.ShapeDtypeStruct((n, DIM), jnp.float32),
        mesh=mesh,
        scratch_types=[
            pltpu.VMEM((chunks, CHUNK), jnp.int32),
            pltpu.VMEM((chunks, CHUNK), jnp.int32),
            pltpu.VMEM((2, DIM), jnp.float32),
            pltpu.VMEM((CHUNK, DIM), jnp.float32),
            pltpu.VMEM((CHUNK, DIM), jnp.float32),
            pltpu.VMEM((CHUNK, DIM), jnp.float32),
            pltpu.VMEM((CHUNK, DIM), jnp.float32),
            pltpu.VMEM((CHUNK, DIM), jnp.float32),
            pltpu.VMEM((CHUNK, DIM), jnp.float32),
            pltpu.SemaphoreType.DMA,
            pltpu.SemaphoreType.DMA,
            pltpu.SemaphoreType.DMA,
            pltpu.SemaphoreType.DMA,
        ],
        compiler_params=pltpu.CompilerParams(
            use_tc_tiling_on_sc=False, needs_layout_passes=False),
    )
    return f(i0, i1, t0, t1, gamma, beta)


def kernel(idx0, idx1, table0, table1, gamma, beta):
    b, f = idx0.shape
    n = b * f
    i0 = idx0.reshape(n // CHUNK, CHUNK).astype(jnp.int32)
    i1 = idx1.reshape(n // CHUNK, CHUNK).astype(jnp.int32)
    out = _fused_sc(i0, i1, table0, table1, gamma, beta, n)
    return out.reshape(b, f, DIM)


# speedup vs baseline: 2.0462x; 1.4827x over previous
import jax
import jax.numpy as jnp
from jax import lax
from jax.experimental import pallas as pl
from jax.experimental.pallas import tpu as pltpu
from jax.experimental.pallas import tpu_sc as plsc

DIM = 64
EPS = 1e-5
NC, NS = 2, 16
NW = NC * NS
CHUNK = 128
NQ = DIM // 16


def _rsqrt_newton(x):
    i = lax.bitcast_convert_type(x, jnp.int32)
    i = jnp.int32(0x5F3759DF) - (i >> 1)
    y = lax.bitcast_convert_type(i, jnp.float32)
    for _ in range(3):
        y = y * (1.5 - 0.5 * x * y * y)
    return y


def _fused_sc(i0, i1, t0, t1, gamma, beta, n):
    per_w = n // NW
    chunks = per_w // CHUNK
    assert chunks % 2 == 0
    mesh = plsc.VectorSubcoreMesh(core_axis_name="c", subcore_axis_name="s")

    def body(i0_hbm, i1_hbm, t0_hbm, t1_hbm, g_hbm, b_hbm, out_hbm,
             i0_v, i1_v, gb_v,
             r0a, r1a, r0b, r1b, oa, ob,
             sga, sgb, soa, sob):
        wid = lax.axis_index("s") * NC + lax.axis_index("c")
        base = wid * per_w
        irow = wid * chunks

        pltpu.sync_copy(i0_hbm.at[pl.ds(irow, chunks)], i0_v)
        pltpu.sync_copy(i1_hbm.at[pl.ds(irow, chunks)], i1_v)
        pltpu.sync_copy(g_hbm, gb_v.at[0])
        pltpu.sync_copy(b_hbm, gb_v.at[1])
        gv = [gb_v[0, pl.ds(16 * q, 16)] for q in range(NQ)]
        bv = [gb_v[1, pl.ds(16 * q, 16)] for q in range(NQ)]

        def issue_gathers(j, r0x, r1x, sgx):
            pltpu.async_copy(t0_hbm.at[i0_v.at[j]], r0x, sgx)
            pltpu.async_copy(t1_hbm.at[i1_v.at[j]], r1x, sgx)

        def wait_gathers(r0x, r1x, sgx):
            pltpu.make_async_copy(t0_hbm.at[i0_v.at[0]], r0x, sgx).wait()
            pltpu.make_async_copy(t1_hbm.at[i1_v.at[0]], r1x, sgx).wait()

        def out_dst(j):
            return out_hbm.at[pl.ds(base + j * CHUNK, CHUNK)]

        def compute(r0x, r1x, ox):
            @plsc.parallel_loop(0, CHUNK, 1, unroll=8)
            def row(k):
                a = [r0x[k, pl.ds(16 * q, 16)] + r1x[k, pl.ds(16 * q, 16)]
                     for q in range(NQ)]
                tot = jnp.sum((a[0] + a[1]) + (a[2] + a[3]))
                tot2 = jnp.sum((a[0] * a[0] + a[1] * a[1])
                               + (a[2] * a[2] + a[3] * a[3]))
                mu = tot * (1.0 / DIM)
                var = tot2 * (1.0 / DIM) - mu * mu
                rstd = _rsqrt_newton(var + EPS)
                for q in range(NQ):
                    ox[k, pl.ds(16 * q, 16)] = (a[q] - mu) * (rstd * gv[q]) + bv[q]

        issue_gathers(0, r0a, r1a, sga)
        pltpu.async_copy(oa, out_dst(0), soa)
        pltpu.async_copy(ob, out_dst(1), sob)

        def pair(p, carry):
            ja = 2 * p
            wait_gathers(r0a, r1a, sga)
            issue_gathers(ja + 1, r0b, r1b, sgb)
            pltpu.make_async_copy(oa, out_dst(0), soa).wait()
            compute(r0a, r1a, oa)
            pltpu.async_copy(oa, out_dst(ja), soa)
            wait_gathers(r0b, r1b, sgb)

            @pl.when(p < chunks // 2 - 1)
            def _():
                issue_gathers(ja + 2, r0a, r1a, sga)

            pltpu.make_async_copy(ob, out_dst(0), sob).wait()
            compute(r0b, r1b, ob)
            pltpu.async_copy(ob, out_dst(ja + 1), sob)
            return carry

        lax.fori_loop(0, chunks // 2, pair, 0)
        pltpu.make_async_copy(oa, out_dst(0), soa).wait()
        pltpu.make_async_copy(ob, out_dst(0), sob).wait()

    f = pl.kernel(
        body,
        out_type=jax.ShapeDtypeStruct((n, DIM), jnp.float32),
        mesh=mesh,
        scratch_types=[
            pltpu.VMEM((chunks, CHUNK), jnp.int32),
            pltpu.VMEM((chunks, CHUNK), jnp.int32),
            pltpu.VMEM((2, DIM), jnp.float32),
            pltpu.VMEM((CHUNK, DIM), jnp.float32),
            pltpu.VMEM((CHUNK, DIM), jnp.float32),
            pltpu.VMEM((CHUNK, DIM), jnp.float32),
            pltpu.VMEM((CHUNK, DIM), jnp.float32),
            pltpu.VMEM((CHUNK, DIM), jnp.float32),
            pltpu.VMEM((CHUNK, DIM), jnp.float32),
            pltpu.SemaphoreType.DMA,
            pltpu.SemaphoreType.DMA,
            pltpu.SemaphoreType.DMA,
            pltpu.SemaphoreType.DMA,
        ],
        compiler_params=pltpu.CompilerParams(
            use_tc_tiling_on_sc=False, needs_layout_passes=False),
    )
    return f(i0, i1, t0, t1, gamma, beta)


def kernel(idx0, idx1, table0, table1, gamma, beta):
    b, f = idx0.shape
    n = b * f
    i0 = idx0.reshape(n // CHUNK, CHUNK).astype(jnp.int32)
    i1 = idx1.reshape(n // CHUNK, CHUNK).astype(jnp.int32)
    out = _fused_sc(i0, i1, table0, table1, gamma, beta, n)
    return out.reshape(b, f, DIM)
